# Initial kernel scaffold; baseline (speedup 1.0000x reference)
import jax
import jax.numpy as jnp
from jax.experimental import pallas as pl


def kernel(x_0, x_1, x_2, adjacency_0, adjacency_1, coadjacency_2, incidence_1, incidence_2, w_hbs0_l1, a_hbs0_l1, ws_01_l1, wt_01_l1, a_01_l1, ws_12_l1, wt_12_l1, a_12_l1, w_hbs0_l2, a_hbs0_l2, ws_01_l2, wt_01_l2, a_01_l2, w_hbs1_l2, a_hbs1_l2, ws_12_l2, wt_12_l2, a_12_l2, w_hbs2_l2, a_hbs2_l2):
    raise NotImplementedError("write your pallas kernel here")


# speedup vs baseline: 2.7228x; 2.7228x over previous
import functools

import jax
import jax.numpy as jnp
from jax import lax
from jax.experimental import pallas as pl
from jax.experimental.pallas import tpu as pltpu
from jax.experimental.pallas import tpu_sc as plsc

_N0, _N1, _N2, _D = 10000, 30000, 20000, 128
_NEG = 0.2
_NW = 16
_CH1 = 256
_CH2 = 128
_EGRAN = _NW * _CH1


def _pad_len(n):
    return ((n + 1 + 255) // 256) * 256


def _pad_edges(e):
    return ((e + _EGRAN - 1) // _EGRAN) * _EGRAN


def _mm_body(x_ref, w_ref, ap_ref, m_ref, s_ref):
    m = jnp.dot(x_ref[...], w_ref[...], preferred_element_type=jnp.float32)
    m_ref[...] = m
    s_ref[...] = jnp.dot(m, ap_ref[...], preferred_element_type=jnp.float32)


def _dense(x, w, a):
    n = x.shape[0]
    blk = 1000
    ap = jnp.pad(jnp.concatenate([a[:_D], a[_D:]], axis=1), ((0, 0), (0, 6)))
    m, s = pl.pallas_call(
        _mm_body,
        grid=(n // blk,),
        in_specs=[pl.BlockSpec((blk, _D), lambda i: (i, 0)),
                  pl.BlockSpec((_D, _D), lambda i: (0, 0)),
                  pl.BlockSpec((_D, 8), lambda i: (0, 0))],
        out_specs=[pl.BlockSpec((blk, _D), lambda i: (i, 0)),
                   pl.BlockSpec((blk, 8), lambda i: (i, 0))],
        out_shape=[jax.ShapeDtypeStruct((n, _D), jnp.float32),
                   jax.ShapeDtypeStruct((n, 8), jnp.float32)],
    )(x, w, ap)
    return m, s[:, 0], s[:, 1]


@functools.lru_cache(maxsize=None)
def _make_scores(n_t_pad, n_s_pad, e_pad):
    iters = e_pad // (_NW * _CH1)
    mesh = plsc.VectorSubcoreMesh(core_axis_name="c", subcore_axis_name="s",
                                  num_cores=1)

    def body(st_hbm, ss_hbm, ii_hbm, jj_hbm, w_hbm, dt_hbm, ds_hbm,
             st_v, ss_v, dt_v, ds_v, iv, jv, wv, sem):
        del sem
        wid = lax.axis_index("s")
        pltpu.sync_copy(st_hbm, st_v)
        pltpu.sync_copy(ss_hbm, ss_v)
        zero16 = jnp.zeros((16,), jnp.float32)

        def z_t(k, c):
            dt_v[pl.ds(k * 16, 16)] = zero16
            return c
        lax.fori_loop(0, n_t_pad // 16, z_t, 0)

        def z_s(k, c):
            ds_v[pl.ds(k * 16, 16)] = zero16
            return c
        lax.fori_loop(0, n_s_pad // 16, z_s, 0)

        def eloop(g, c):
            base = (g * _NW + wid) * _CH1
            pltpu.sync_copy(ii_hbm.at[pl.ds(base, _CH1)], iv)
            pltpu.sync_copy(jj_hbm.at[pl.ds(base, _CH1)], jv)

            def chunk(k, c2):
                i16 = iv[pl.ds(k * 16, 16)]
                j16 = jv[pl.ds(k * 16, 16)]
                e = plsc.load_gather(st_v, [i16]) + plsc.load_gather(ss_v, [j16])
                e = jnp.where(e >= 0, e, _NEG * e)
                w16 = jnp.exp(e)
                wv[pl.ds(k * 16, 16)] = w16
                plsc.addupdate_scatter(dt_v, [i16], w16)
                plsc.addupdate_scatter(ds_v, [j16], w16)
                return c2
            lax.fori_loop(0, _CH1 // 16, chunk, 0)
            pltpu.sync_copy(wv, w_hbm.at[pl.ds(base, _CH1)])
            return c
        lax.fori_loop(0, iters, eloop, 0)
        pltpu.sync_copy(dt_v, dt_hbm.at[wid])
        pltpu.sync_copy(ds_v, ds_hbm.at[wid])

    return pl.kernel(
        body, mesh=mesh,
        compiler_params=pltpu.CompilerParams(needs_layout_passes=False),
        out_type=[jax.ShapeDtypeStruct((e_pad,), jnp.float32),
                  jax.ShapeDtypeStruct((_NW, n_t_pad), jnp.float32),
                  jax.ShapeDtypeStruct((_NW, n_s_pad), jnp.float32)],
        scratch_types=[pltpu.VMEM((n_t_pad,), jnp.float32),
                       pltpu.VMEM((n_s_pad,), jnp.float32),
                       pltpu.VMEM((n_t_pad,), jnp.float32),
                       pltpu.VMEM((n_s_pad,), jnp.float32),
                       pltpu.VMEM((_CH1,), jnp.int32),
                       pltpu.VMEM((_CH1,), jnp.int32),
                       pltpu.VMEM((_CH1,), jnp.float32),
                       pltpu.SemaphoreType.DMA],
    )


@functools.lru_cache(maxsize=None)
def _make_aggregate(n_t_pad, n_s_pad, e_pad):
    del n_s_pad
    iters = e_pad // (_NW * _CH2)
    mesh = plsc.VectorSubcoreMesh(core_axis_name="c", subcore_axis_name="s",
                                  num_cores=1)

    def body(m0, m1, m2, m3, di_hbm, sj_hbm, w_hbm, zr_hbm,
             a0, a1, a2, a3, dv, sv, wv, rows, shacc, sem):
        wid = lax.axis_index("s")
        msrc = (m0, m1, m2, m3)
        outs = (a0, a1, a2, a3)
        sl = n_t_pad // _NW
        for cc in range(4):
            @pl.when(wid == 0)
            def _():
                pltpu.sync_copy(zr_hbm, shacc)
            plsc.subcore_barrier()

            def eloop(g, c):
                base = (g * _NW + wid) * _CH2
                pltpu.sync_copy(di_hbm.at[pl.ds(base, _CH2)], dv)
                pltpu.sync_copy(sj_hbm.at[pl.ds(base, _CH2)], sv)
                pltpu.sync_copy(w_hbm.at[pl.ds(base, _CH2)], wv)
                pltpu.async_copy(msrc[cc].at[sv], rows, sem).wait()

                def scale(k, c2):
                    w16 = wv[pl.ds(k * 16, 16)]
                    for t in range(16):
                        wsc = w16[t]
                        e = k * 16 + t
                        rows[e, pl.ds(0, 16)] = rows[e, pl.ds(0, 16)] * wsc
                        rows[e, pl.ds(16, 16)] = rows[e, pl.ds(16, 16)] * wsc
                    return c2
                lax.fori_loop(0, _CH2 // 16, scale, 0)
                pltpu.sync_copy(rows, shacc.at[dv], add=True)
                return c
            lax.fori_loop(0, iters, eloop, 0)
            plsc.subcore_barrier()
            lo = wid * sl
            pltpu.sync_copy(shacc.at[pl.ds(lo, sl)], outs[cc].at[pl.ds(lo, sl)])
            plsc.subcore_barrier()

    return pl.kernel(
        body, mesh=mesh,
        compiler_params=pltpu.CompilerParams(needs_layout_passes=False,
                                             use_tc_tiling_on_sc=False),
        out_type=[jax.ShapeDtypeStruct((n_t_pad, 32), jnp.float32)] * 4,
        scratch_types=[pltpu.VMEM((_CH2,), jnp.int32),
                       pltpu.VMEM((_CH2,), jnp.int32),
                       pltpu.VMEM((_CH2,), jnp.float32),
                       pltpu.VMEM((_CH2, 32), jnp.float32),
                       pltpu.VMEM_SHARED((n_t_pad, 32), jnp.float32),
                       pltpu.SemaphoreType.DMA],
    )


def _edge_scores(st, ss, ii, jj, n_t, n_s):
    n_t_pad, n_s_pad = _pad_len(n_t), _pad_len(n_s)
    e_pad = _pad_edges(ii.shape[0])
    stp = jnp.pad(st, (0, n_t_pad - n_t))
    ssp = jnp.pad(ss, (0, n_s_pad - n_s))
    iip = jnp.pad(ii, (0, e_pad - ii.shape[0]), constant_values=n_t)
    jjp = jnp.pad(jj, (0, e_pad - jj.shape[0]), constant_values=n_s)
    w, dtp, dsp = _make_scores(n_t_pad, n_s_pad, e_pad)(stp, ssp, iip, jjp)
    return w, dtp, dsp, iip, jjp


def _edge_aggregate(msg, w, di, sj, n_t, n_s):
    n_t_pad, n_s_pad = _pad_len(n_t), _pad_len(n_s)
    mp = jnp.pad(msg, ((0, n_s_pad - n_s), (0, 0)))
    parts = [mp[:, k * 32:(k + 1) * 32] for k in range(4)]
    zr = jnp.zeros((n_t_pad, 32), jnp.float32)
    a0, a1, a2, a3 = _make_aggregate(n_t_pad, n_s_pad, di.shape[0])(
        parts[0], parts[1], parts[2], parts[3], di, sj, w, zr)
    return jnp.concatenate([a0, a1, a2, a3], axis=1)[:n_t]


def _combine(terms, n):
    blk = 1000
    k = len(terms)

    def body(*refs):
        out = jnp.zeros((blk, _D), jnp.float32)
        for t in range(k):
            a = refs[2 * t][...]
            d = jnp.sum(refs[2 * t + 1][...], axis=1, keepdims=True)
            d = jnp.where(d == 0., 1., d)
            out = out + jnp.maximum(a / d, 0.)
        refs[-1][...] = out

    in_specs = []
    args = []
    for (acc, dp) in terms:
        in_specs.append(pl.BlockSpec((blk, _D), lambda i: (i, 0)))
        in_specs.append(pl.BlockSpec((blk, _NW), lambda i: (i, 0)))
        args.append(acc)
        args.append(dp.T[:n])
    return pl.pallas_call(
        body,
        grid=(n // blk,),
        in_specs=in_specs,
        out_specs=pl.BlockSpec((blk, _D), lambda i: (i, 0)),
        out_shape=jax.ShapeDtypeStruct((n, _D), jnp.float32),
    )(*args)


def _hbs(x, idx, w, a, n):
    m, s_top, s_bot = _dense(x, w, a)
    ii, jj = idx[0], idx[1]
    wv, dtp, _, iip, jjp = _edge_scores(s_top, s_bot, ii, jj, n, n)
    acc = _edge_aggregate(m, wv, iip, jjp, n, n)
    return acc, dtp


def _hbns(x_s, x_t, idx, w_s, w_t, a, n_s, n_t):
    sm, sm_top, _ = _dense(x_s, w_s, a)
    tm, _, tm_bot = _dense(x_t, w_t, a)
    ii, jj = idx[0], idx[1]
    wv, dtp, dsp, iip, jjp = _edge_scores(tm_bot, sm_top, ii, jj, n_t, n_s)
    acc_t = _edge_aggregate(sm, wv, iip, jjp, n_t, n_s)
    acc_s = _edge_aggregate(tm, wv, jjp, iip, n_s, n_t)
    return (acc_s, dsp), (acc_t, dtp)


def kernel(x_0, x_1, x_2, adjacency_0, adjacency_1, coadjacency_2,
           incidence_1, incidence_2,
           w_hbs0_l1, a_hbs0_l1, ws_01_l1, wt_01_l1, a_01_l1,
           ws_12_l1, wt_12_l1, a_12_l1,
           w_hbs0_l2, a_hbs0_l2, ws_01_l2, wt_01_l2, a_01_l2,
           w_hbs1_l2, a_hbs1_l2, ws_12_l2, wt_12_l2, a_12_l2,
           w_hbs2_l2, a_hbs2_l2):
    x00, d00 = _hbs(x_0, adjacency_0, w_hbs0_l1, a_hbs0_l1, _N0)
    (x01, d01), (x10, d10) = _hbns(x_1, x_0, incidence_1,
                                   ws_01_l1, wt_01_l1, a_01_l1, _N1, _N0)
    (x12, d12), (x21, d21) = _hbns(x_2, x_1, incidence_2,
                                   ws_12_l1, wt_12_l1, a_12_l1, _N2, _N1)
    x0l1 = _combine([(x00, d00), (x10, d10)], _N0)
    x1l1 = _combine([(x01, d01), (x21, d21)], _N1)
    x2l1 = _combine([(x12, d12)], _N2)

    y00, e00 = _hbs(x0l1, adjacency_0, w_hbs0_l2, a_hbs0_l2, _N0)
    (y01, e01), (y10, e10) = _hbns(x1l1, x0l1, incidence_1,
                                   ws_01_l2, wt_01_l2, a_01_l2, _N1, _N0)
    y11, e11 = _hbs(x1l1, adjacency_1, w_hbs1_l2, a_hbs1_l2, _N1)
    (y12, e12), (y21, e21) = _hbns(x2l1, x1l1, incidence_2,
                                   ws_12_l2, wt_12_l2, a_12_l2, _N2, _N1)
    y22, e22 = _hbs(x2l1, coadjacency_2, w_hbs2_l2, a_hbs2_l2, _N2)
    x0l2 = _combine([(y00, e00), (y10, e10)], _N0)
    x1l2 = _combine([(y01, e01), (y11, e11), (y21, e21)], _N1)
    x2l2 = _combine([(y12, e12), (y22, e22)], _N2)
    return x0l2, x1l2, x2l2


# speedup vs baseline: 3.2883x; 1.2077x over previous
import functools

import jax
import jax.numpy as jnp
from jax import lax
from jax.experimental import pallas as pl
from jax.experimental.pallas import tpu as pltpu
from jax.experimental.pallas import tpu_sc as plsc

_N0, _N1, _N2, _D = 10000, 30000, 20000, 128
_NEG = 0.2
_NC = 2
_NS = 16
_NW = _NC * _NS
_CH1 = 256
_CH2 = 128
_EGRAN = _NW * _CH1


def _pad_len(n):
    return ((n + 1 + 255) // 256) * 256


def _pad_edges(e):
    return ((e + _EGRAN - 1) // _EGRAN) * _EGRAN


def _mm_body(x_ref, w_ref, ap_ref, m_ref, s_ref):
    m = jnp.dot(x_ref[...], w_ref[...], preferred_element_type=jnp.float32)
    m_ref[...] = m
    s_ref[...] = jnp.dot(m, ap_ref[...], preferred_element_type=jnp.float32)


def _dense(x, w, a):
    n = x.shape[0]
    blk = 1000
    ap = jnp.pad(jnp.concatenate([a[:_D], a[_D:]], axis=1), ((0, 0), (0, 6)))
    m, s = pl.pallas_call(
        _mm_body,
        grid=(n // blk,),
        in_specs=[pl.BlockSpec((blk, _D), lambda i: (i, 0)),
                  pl.BlockSpec((_D, _D), lambda i: (0, 0)),
                  pl.BlockSpec((_D, 8), lambda i: (0, 0))],
        out_specs=[pl.BlockSpec((blk, _D), lambda i: (i, 0)),
                   pl.BlockSpec((blk, 8), lambda i: (i, 0))],
        out_shape=[jax.ShapeDtypeStruct((n, _D), jnp.float32),
                   jax.ShapeDtypeStruct((n, 8), jnp.float32)],
    )(x, w, ap)
    return m, s[:, 0], s[:, 1]


@functools.lru_cache(maxsize=None)
def _make_scores(n_t_pad, n_s_pad, e_pad):
    iters = e_pad // (_NW * _CH1)
    mesh = plsc.VectorSubcoreMesh(core_axis_name="c", subcore_axis_name="s",
                                  num_cores=_NC)

    def body(st_hbm, ss_hbm, ii_hbm, jj_hbm, w_hbm, dt_hbm, ds_hbm,
             st_v, ss_v, dt_v, ds_v, iv, jv, wv, sem):
        del sem
        wid = lax.axis_index("s") * _NC + lax.axis_index("c")
        pltpu.sync_copy(st_hbm, st_v)
        pltpu.sync_copy(ss_hbm, ss_v)
        zero16 = jnp.zeros((16,), jnp.float32)

        def z_t(k, c):
            dt_v[pl.ds(k * 16, 16)] = zero16
            return c
        lax.fori_loop(0, n_t_pad // 16, z_t, 0)

        def z_s(k, c):
            ds_v[pl.ds(k * 16, 16)] = zero16
            return c
        lax.fori_loop(0, n_s_pad // 16, z_s, 0)

        def eloop(g, c):
            base = (g * _NW + wid) * _CH1
            pltpu.sync_copy(ii_hbm.at[pl.ds(base, _CH1)], iv)
            pltpu.sync_copy(jj_hbm.at[pl.ds(base, _CH1)], jv)

            def chunk(k, c2):
                i16 = iv[pl.ds(k * 16, 16)]
                j16 = jv[pl.ds(k * 16, 16)]
                e = plsc.load_gather(st_v, [i16]) + plsc.load_gather(ss_v, [j16])
                e = jnp.where(e >= 0, e, _NEG * e)
                w16 = jnp.exp(e)
                wv[pl.ds(k * 16, 16)] = w16
                plsc.addupdate_scatter(dt_v, [i16], w16)
                plsc.addupdate_scatter(ds_v, [j16], w16)
                return c2
            lax.fori_loop(0, _CH1 // 16, chunk, 0)
            pltpu.sync_copy(wv, w_hbm.at[pl.ds(base, _CH1)])
            return c
        lax.fori_loop(0, iters, eloop, 0)
        pltpu.sync_copy(dt_v, dt_hbm.at[wid])
        pltpu.sync_copy(ds_v, ds_hbm.at[wid])

    return pl.kernel(
        body, mesh=mesh,
        compiler_params=pltpu.CompilerParams(needs_layout_passes=False),
        out_type=[jax.ShapeDtypeStruct((e_pad,), jnp.float32),
                  jax.ShapeDtypeStruct((_NW, n_t_pad), jnp.float32),
                  jax.ShapeDtypeStruct((_NW, n_s_pad), jnp.float32)],
        scratch_types=[pltpu.VMEM((n_t_pad,), jnp.float32),
                       pltpu.VMEM((n_s_pad,), jnp.float32),
                       pltpu.VMEM((n_t_pad,), jnp.float32),
                       pltpu.VMEM((n_s_pad,), jnp.float32),
                       pltpu.VMEM((_CH1,), jnp.int32),
                       pltpu.VMEM((_CH1,), jnp.int32),
                       pltpu.VMEM((_CH1,), jnp.float32),
                       pltpu.SemaphoreType.DMA],
    )


@functools.lru_cache(maxsize=None)
def _make_aggregate(n_t_pad, n_s_pad, e_pad):
    del n_s_pad
    iters = e_pad // (_NW * _CH2)
    mesh = plsc.VectorSubcoreMesh(core_axis_name="c", subcore_axis_name="s",
                                  num_cores=_NC)

    def body(m0, m1, m2, m3, di_hbm, sj_hbm, w_hbm, zr_hbm,
             a0, a1, a2, a3, dv, sv, wv, rows, shacc, sem):
        sid = lax.axis_index("s")
        cid = lax.axis_index("c")
        wid = sid * _NC + cid
        msrc = (m0, m1, m2, m3)
        outs = (a0, a1, a2, a3)
        sl = n_t_pad // _NS
        for cc in range(4):
            @pl.when(sid == 0)
            def _():
                pltpu.sync_copy(zr_hbm, shacc)
            plsc.subcore_barrier()

            def eloop(g, c):
                base = (g * _NW + wid) * _CH2
                pltpu.sync_copy(di_hbm.at[pl.ds(base, _CH2)], dv)
                pltpu.sync_copy(sj_hbm.at[pl.ds(base, _CH2)], sv)
                pltpu.sync_copy(w_hbm.at[pl.ds(base, _CH2)], wv)
                pltpu.async_copy(msrc[cc].at[sv], rows, sem).wait()

                def scale(k, c2):
                    w16 = wv[pl.ds(k * 16, 16)]
                    for t in range(16):
                        wsc = w16[t]
                        e = k * 16 + t
                        rows[e, pl.ds(0, 16)] = rows[e, pl.ds(0, 16)] * wsc
                        rows[e, pl.ds(16, 16)] = rows[e, pl.ds(16, 16)] * wsc
                    return c2
                lax.fori_loop(0, _CH2 // 16, scale, 0)
                pltpu.sync_copy(rows, shacc.at[dv], add=True)
                return c
            lax.fori_loop(0, iters, eloop, 0)
            plsc.subcore_barrier()
            lo = sid * sl
            pltpu.sync_copy(shacc.at[pl.ds(lo, sl)],
                            outs[cc].at[cid, pl.ds(lo, sl)])
            plsc.subcore_barrier()

    return pl.kernel(
        body, mesh=mesh,
        compiler_params=pltpu.CompilerParams(needs_layout_passes=False,
                                             use_tc_tiling_on_sc=False),
        out_type=[jax.ShapeDtypeStruct((_NC, n_t_pad, 32), jnp.float32)] * 4,
        scratch_types=[pltpu.VMEM((_CH2,), jnp.int32),
                       pltpu.VMEM((_CH2,), jnp.int32),
                       pltpu.VMEM((_CH2,), jnp.float32),
                       pltpu.VMEM((_CH2, 32), jnp.float32),
                       pltpu.VMEM_SHARED((n_t_pad, 32), jnp.float32),
                       pltpu.SemaphoreType.DMA],
    )


def _edge_scores(st, ss, ii, jj, n_t, n_s):
    n_t_pad, n_s_pad = _pad_len(n_t), _pad_len(n_s)
    e_pad = _pad_edges(ii.shape[0])
    stp = jnp.pad(st, (0, n_t_pad - n_t))
    ssp = jnp.pad(ss, (0, n_s_pad - n_s))
    iip = jnp.pad(ii, (0, e_pad - ii.shape[0]), constant_values=n_t)
    jjp = jnp.pad(jj, (0, e_pad - jj.shape[0]), constant_values=n_s)
    w, dtp, dsp = _make_scores(n_t_pad, n_s_pad, e_pad)(stp, ssp, iip, jjp)
    return w, dtp, dsp, iip, jjp


def _edge_aggregate(msg, w, di, sj, n_t, n_s):
    n_t_pad, n_s_pad = _pad_len(n_t), _pad_len(n_s)
    mp = jnp.pad(msg, ((0, n_s_pad - n_s), (0, 0)))
    parts = [mp[:, k * 32:(k + 1) * 32] for k in range(4)]
    zr = jnp.zeros((n_t_pad, 32), jnp.float32)
    a0, a1, a2, a3 = _make_aggregate(n_t_pad, n_s_pad, di.shape[0])(
        parts[0], parts[1], parts[2], parts[3], di, sj, w, zr)
    acc_a = jnp.concatenate([a0[0], a1[0], a2[0], a3[0]], axis=1)[:n_t]
    acc_b = jnp.concatenate([a0[1], a1[1], a2[1], a3[1]], axis=1)[:n_t]
    return acc_a, acc_b


def _combine(terms, n):
    blk = 1000
    k = len(terms)

    def body(*refs):
        out = jnp.zeros((blk, _D), jnp.float32)
        for t in range(k):
            a = refs[3 * t][...] + refs[3 * t + 1][...]
            d = jnp.sum(refs[3 * t + 2][...], axis=1, keepdims=True)
            d = jnp.where(d == 0., 1., d)
            out = out + jnp.maximum(a / d, 0.)
        refs[-1][...] = out

    in_specs = []
    args = []
    for (acc_a, acc_b, dp) in terms:
        in_specs.append(pl.BlockSpec((blk, _D), lambda i: (i, 0)))
        in_specs.append(pl.BlockSpec((blk, _D), lambda i: (i, 0)))
        in_specs.append(pl.BlockSpec((blk, _NW), lambda i: (i, 0)))
        args.append(acc_a)
        args.append(acc_b)
        args.append(dp.T[:n])
    return pl.pallas_call(
        body,
        grid=(n // blk,),
        in_specs=in_specs,
        out_specs=pl.BlockSpec((blk, _D), lambda i: (i, 0)),
        out_shape=jax.ShapeDtypeStruct((n, _D), jnp.float32),
    )(*args)


def _hbs(x, idx, w, a, n):
    m, s_top, s_bot = _dense(x, w, a)
    ii, jj = idx[0], idx[1]
    wv, dtp, _, iip, jjp = _edge_scores(s_top, s_bot, ii, jj, n, n)
    acc_a, acc_b = _edge_aggregate(m, wv, iip, jjp, n, n)
    return acc_a, acc_b, dtp


def _hbns(x_s, x_t, idx, w_s, w_t, a, n_s, n_t):
    sm, sm_top, _ = _dense(x_s, w_s, a)
    tm, _, tm_bot = _dense(x_t, w_t, a)
    ii, jj = idx[0], idx[1]
    wv, dtp, dsp, iip, jjp = _edge_scores(tm_bot, sm_top, ii, jj, n_t, n_s)
    at_a, at_b = _edge_aggregate(sm, wv, iip, jjp, n_t, n_s)
    as_a, as_b = _edge_aggregate(tm, wv, jjp, iip, n_s, n_t)
    return (as_a, as_b, dsp), (at_a, at_b, dtp)


def kernel(x_0, x_1, x_2, adjacency_0, adjacency_1, coadjacency_2,
           incidence_1, incidence_2,
           w_hbs0_l1, a_hbs0_l1, ws_01_l1, wt_01_l1, a_01_l1,
           ws_12_l1, wt_12_l1, a_12_l1,
           w_hbs0_l2, a_hbs0_l2, ws_01_l2, wt_01_l2, a_01_l2,
           w_hbs1_l2, a_hbs1_l2, ws_12_l2, wt_12_l2, a_12_l2,
           w_hbs2_l2, a_hbs2_l2):
    t00 = _hbs(x_0, adjacency_0, w_hbs0_l1, a_hbs0_l1, _N0)
    t01, t10 = _hbns(x_1, x_0, incidence_1,
                     ws_01_l1, wt_01_l1, a_01_l1, _N1, _N0)
    t12, t21 = _hbns(x_2, x_1, incidence_2,
                     ws_12_l1, wt_12_l1, a_12_l1, _N2, _N1)
    x0l1 = _combine([t00, t10], _N0)
    x1l1 = _combine([t01, t21], _N1)
    x2l1 = _combine([t12], _N2)

    u00 = _hbs(x0l1, adjacency_0, w_hbs0_l2, a_hbs0_l2, _N0)
    u01, u10 = _hbns(x1l1, x0l1, incidence_1,
                     ws_01_l2, wt_01_l2, a_01_l2, _N1, _N0)
    u11 = _hbs(x1l1, adjacency_1, w_hbs1_l2, a_hbs1_l2, _N1)
    u12, u21 = _hbns(x2l1, x1l1, incidence_2,
                     ws_12_l2, wt_12_l2, a_12_l2, _N2, _N1)
    u22 = _hbs(x2l1, coadjacency_2, w_hbs2_l2, a_hbs2_l2, _N2)
    x0l2 = _combine([u00, u10], _N0)
    x1l2 = _combine([u01, u11, u21], _N1)
    x2l2 = _combine([u12, u22], _N2)
    return x0l2, x1l2, x2l2


# speedup vs baseline: 3.8677x; 1.1762x over previous
import functools

import jax
import jax.numpy as jnp
from jax import lax
from jax.experimental import pallas as pl
from jax.experimental.pallas import tpu as pltpu
from jax.experimental.pallas import tpu_sc as plsc

_N0, _N1, _N2, _D = 10000, 30000, 20000, 128
_NEG = 0.2
_NC = 2
_NS = 16
_NW = _NC * _NS
_CH1 = 256
_CH2 = 256
_EGRAN = _NW * _CH1


def _pad_len(n):
    return ((n + 1 + 255) // 256) * 256


def _pad_edges(e):
    return ((e + _EGRAN - 1) // _EGRAN) * _EGRAN


def _mm_body(x_ref, w_ref, ap_ref, m_ref, s_ref):
    m = jnp.dot(x_ref[...], w_ref[...], preferred_element_type=jnp.float32)
    m_ref[...] = m
    s_ref[...] = jnp.dot(m, ap_ref[...], preferred_element_type=jnp.float32)


def _dense(x, w, a):
    n = x.shape[0]
    blk = 1000
    ap = jnp.pad(jnp.concatenate([a[:_D], a[_D:]], axis=1), ((0, 0), (0, 6)))
    m, s = pl.pallas_call(
        _mm_body,
        grid=(n // blk,),
        in_specs=[pl.BlockSpec((blk, _D), lambda i: (i, 0)),
                  pl.BlockSpec((_D, _D), lambda i: (0, 0)),
                  pl.BlockSpec((_D, 8), lambda i: (0, 0))],
        out_specs=[pl.BlockSpec((blk, _D), lambda i: (i, 0)),
                   pl.BlockSpec((blk, 8), lambda i: (i, 0))],
        out_shape=[jax.ShapeDtypeStruct((n, _D), jnp.float32),
                   jax.ShapeDtypeStruct((n, 8), jnp.float32)],
    )(x, w, ap)
    return m, s[:, 0], s[:, 1]


@functools.lru_cache(maxsize=None)
def _make_scores(n_t_pad, n_s_pad, e_pad):
    iters = e_pad // (_NW * _CH1)
    mesh = plsc.VectorSubcoreMesh(core_axis_name="c", subcore_axis_name="s",
                                  num_cores=_NC)

    def body(st_hbm, ss_hbm, ii_hbm, jj_hbm, w_hbm, dt_hbm, ds_hbm,
             st_v, ss_v, dt_v, ds_v, iv, jv, wv, sem):
        del sem
        wid = lax.axis_index("s") * _NC + lax.axis_index("c")
        pltpu.sync_copy(st_hbm, st_v)
        pltpu.sync_copy(ss_hbm, ss_v)
        zero16 = jnp.zeros((16,), jnp.float32)

        def z_t(k, c):
            dt_v[pl.ds(k * 16, 16)] = zero16
            return c
        lax.fori_loop(0, n_t_pad // 16, z_t, 0)

        def z_s(k, c):
            ds_v[pl.ds(k * 16, 16)] = zero16
            return c
        lax.fori_loop(0, n_s_pad // 16, z_s, 0)

        def eloop(g, c):
            base = (g * _NW + wid) * _CH1
            pltpu.sync_copy(ii_hbm.at[pl.ds(base, _CH1)], iv)
            pltpu.sync_copy(jj_hbm.at[pl.ds(base, _CH1)], jv)

            def chunk(k, c2):
                i16 = iv[pl.ds(k * 16, 16)]
                j16 = jv[pl.ds(k * 16, 16)]
                e = plsc.load_gather(st_v, [i16]) + plsc.load_gather(ss_v, [j16])
                e = jnp.where(e >= 0, e, _NEG * e)
                w16 = jnp.exp(e)
                wv[pl.ds(k * 16, 16)] = w16
                plsc.addupdate_scatter(dt_v, [i16], w16)
                plsc.addupdate_scatter(ds_v, [j16], w16)
                return c2
            lax.fori_loop(0, _CH1 // 16, chunk, 0)
            pltpu.sync_copy(wv, w_hbm.at[pl.ds(base, _CH1)])
            return c
        lax.fori_loop(0, iters, eloop, 0)
        pltpu.sync_copy(dt_v, dt_hbm.at[wid])
        pltpu.sync_copy(ds_v, ds_hbm.at[wid])

    return pl.kernel(
        body, mesh=mesh,
        compiler_params=pltpu.CompilerParams(needs_layout_passes=False),
        out_type=[jax.ShapeDtypeStruct((e_pad,), jnp.float32),
                  jax.ShapeDtypeStruct((_NW, n_t_pad), jnp.float32),
                  jax.ShapeDtypeStruct((_NW, n_s_pad), jnp.float32)],
        scratch_types=[pltpu.VMEM((n_t_pad,), jnp.float32),
                       pltpu.VMEM((n_s_pad,), jnp.float32),
                       pltpu.VMEM((n_t_pad,), jnp.float32),
                       pltpu.VMEM((n_s_pad,), jnp.float32),
                       pltpu.VMEM((_CH1,), jnp.int32),
                       pltpu.VMEM((_CH1,), jnp.int32),
                       pltpu.VMEM((_CH1,), jnp.float32),
                       pltpu.SemaphoreType.DMA],
    )


@functools.lru_cache(maxsize=None)
def _make_aggregate(n_t_pad, n_s_pad, e_pad):
    del n_s_pad
    iters = e_pad // (_NW * _CH2)
    mesh = plsc.VectorSubcoreMesh(core_axis_name="c", subcore_axis_name="s",
                                  num_cores=_NC)

    def body(m0, m1, m2, m3, di_hbm, sj_hbm, w_hbm, zr_hbm,
             a0, a1, a2, a3, dv, sv, wv, rows, shacc, sem):
        sid = lax.axis_index("s")
        cid = lax.axis_index("c")
        wid = sid * _NC + cid
        msrc = (m0, m1, m2, m3)
        outs = (a0, a1, a2, a3)
        sl = n_t_pad // _NS
        for cc in range(4):
            @pl.when(sid == 0)
            def _():
                pltpu.sync_copy(zr_hbm, shacc)
            plsc.subcore_barrier()

            def eloop(g, c):
                base = (g * _NW + wid) * _CH2
                pltpu.sync_copy(di_hbm.at[pl.ds(base, _CH2)], dv)
                pltpu.sync_copy(sj_hbm.at[pl.ds(base, _CH2)], sv)
                pltpu.sync_copy(w_hbm.at[pl.ds(base, _CH2)], wv)
                pltpu.async_copy(msrc[cc].at[sv], rows, sem).wait()

                def scale(k, c2):
                    w16 = wv[pl.ds(k * 16, 16)]
                    for t in range(16):
                        wsc = w16[t]
                        e = k * 16 + t
                        rows[e, pl.ds(0, 16)] = rows[e, pl.ds(0, 16)] * wsc
                        rows[e, pl.ds(16, 16)] = rows[e, pl.ds(16, 16)] * wsc
                    return c2
                lax.fori_loop(0, _CH2 // 16, scale, 0)
                pltpu.sync_copy(rows, shacc.at[dv], add=True)
                return c
            lax.fori_loop(0, iters, eloop, 0)
            plsc.subcore_barrier()
            lo = sid * sl
            pltpu.sync_copy(shacc.at[pl.ds(lo, sl)],
                            outs[cc].at[cid, pl.ds(lo, sl)])
            plsc.subcore_barrier()

    return pl.kernel(
        body, mesh=mesh,
        compiler_params=pltpu.CompilerParams(needs_layout_passes=False,
                                             use_tc_tiling_on_sc=False),
        out_type=[jax.ShapeDtypeStruct((_NC, n_t_pad, 32), jnp.float32)] * 4,
        scratch_types=[pltpu.VMEM((_CH2,), jnp.int32),
                       pltpu.VMEM((_CH2,), jnp.int32),
                       pltpu.VMEM((_CH2,), jnp.float32),
                       pltpu.VMEM((_CH2, 32), jnp.float32),
                       pltpu.VMEM_SHARED((n_t_pad, 32), jnp.float32),
                       pltpu.SemaphoreType.DMA],
    )


def _edge_scores(st, ss, ii, jj, n_t, n_s):
    n_t_pad, n_s_pad = _pad_len(n_t), _pad_len(n_s)
    e_pad = _pad_edges(ii.shape[0])
    stp = jnp.pad(st, (0, n_t_pad - n_t))
    ssp = jnp.pad(ss, (0, n_s_pad - n_s))
    iip = jnp.pad(ii, (0, e_pad - ii.shape[0]), constant_values=n_t)
    jjp = jnp.pad(jj, (0, e_pad - jj.shape[0]), constant_values=n_s)
    w, dtp, dsp = _make_scores(n_t_pad, n_s_pad, e_pad)(stp, ssp, iip, jjp)
    return w, dtp, dsp, iip, jjp


def _edge_aggregate(msg, w, di, sj, n_t, n_s):
    n_t_pad, n_s_pad = _pad_len(n_t), _pad_len(n_s)
    mp = jnp.pad(msg, ((0, n_s_pad - n_s), (0, 0)))
    parts = [mp[:, k * 32:(k + 1) * 32] for k in range(4)]
    zr = jnp.zeros((n_t_pad, 32), jnp.float32)
    a0, a1, a2, a3 = _make_aggregate(n_t_pad, n_s_pad, di.shape[0])(
        parts[0], parts[1], parts[2], parts[3], di, sj, w, zr)
    acc_a = jnp.concatenate([a0[0], a1[0], a2[0], a3[0]], axis=1)[:n_t]
    acc_b = jnp.concatenate([a0[1], a1[1], a2[1], a3[1]], axis=1)[:n_t]
    return acc_a, acc_b


def _combine(terms, n):
    blk = 1000
    k = len(terms)

    def body(*refs):
        out = jnp.zeros((blk, _D), jnp.float32)
        for t in range(k):
            a = refs[3 * t][...] + refs[3 * t + 1][...]
            d = jnp.sum(refs[3 * t + 2][...], axis=1, keepdims=True)
            d = jnp.where(d == 0., 1., d)
            out = out + jnp.maximum(a / d, 0.)
        refs[-1][...] = out

    in_specs = []
    args = []
    for (acc_a, acc_b, dp) in terms:
        in_specs.append(pl.BlockSpec((blk, _D), lambda i: (i, 0)))
        in_specs.append(pl.BlockSpec((blk, _D), lambda i: (i, 0)))
        in_specs.append(pl.BlockSpec((blk, _NW), lambda i: (i, 0)))
        args.append(acc_a)
        args.append(acc_b)
        args.append(dp.T[:n])
    return pl.pallas_call(
        body,
        grid=(n // blk,),
        in_specs=in_specs,
        out_specs=pl.BlockSpec((blk, _D), lambda i: (i, 0)),
        out_shape=jax.ShapeDtypeStruct((n, _D), jnp.float32),
    )(*args)


def _hbs(x, idx, w, a, n):
    m, s_top, s_bot = _dense(x, w, a)
    ii, jj = idx[0], idx[1]
    wv, dtp, _, iip, jjp = _edge_scores(s_top, s_bot, ii, jj, n, n)
    acc_a, acc_b = _edge_aggregate(m, wv, iip, jjp, n, n)
    return acc_a, acc_b, dtp


def _hbns(x_s, x_t, idx, w_s, w_t, a, n_s, n_t):
    sm, sm_top, _ = _dense(x_s, w_s, a)
    tm, _, tm_bot = _dense(x_t, w_t, a)
    ii, jj = idx[0], idx[1]
    wv, dtp, dsp, iip, jjp = _edge_scores(tm_bot, sm_top, ii, jj, n_t, n_s)
    at_a, at_b = _edge_aggregate(sm, wv, iip, jjp, n_t, n_s)
    as_a, as_b = _edge_aggregate(tm, wv, jjp, iip, n_s, n_t)
    return (as_a, as_b, dsp), (at_a, at_b, dtp)


def kernel(x_0, x_1, x_2, adjacency_0, adjacency_1, coadjacency_2,
           incidence_1, incidence_2,
           w_hbs0_l1, a_hbs0_l1, ws_01_l1, wt_01_l1, a_01_l1,
           ws_12_l1, wt_12_l1, a_12_l1,
           w_hbs0_l2, a_hbs0_l2, ws_01_l2, wt_01_l2, a_01_l2,
           w_hbs1_l2, a_hbs1_l2, ws_12_l2, wt_12_l2, a_12_l2,
           w_hbs2_l2, a_hbs2_l2):
    t00 = _hbs(x_0, adjacency_0, w_hbs0_l1, a_hbs0_l1, _N0)
    t01, t10 = _hbns(x_1, x_0, incidence_1,
                     ws_01_l1, wt_01_l1, a_01_l1, _N1, _N0)
    t12, t21 = _hbns(x_2, x_1, incidence_2,
                     ws_12_l1, wt_12_l1, a_12_l1, _N2, _N1)
    x0l1 = _combine([t00, t10], _N0)
    x1l1 = _combine([t01, t21], _N1)
    x2l1 = _combine([t12], _N2)

    u00 = _hbs(x0l1, adjacency_0, w_hbs0_l2, a_hbs0_l2, _N0)
    u01, u10 = _hbns(x1l1, x0l1, incidence_1,
                     ws_01_l2, wt_01_l2, a_01_l2, _N1, _N0)
    u11 = _hbs(x1l1, adjacency_1, w_hbs1_l2, a_hbs1_l2, _N1)
    u12, u21 = _hbns(x2l1, x1l1, incidence_2,
                     ws_12_l2, wt_12_l2, a_12_l2, _N2, _N1)
    u22 = _hbs(x2l1, coadjacency_2, w_hbs2_l2, a_hbs2_l2, _N2)
    x0l2 = _combine([u00, u10], _N0)
    x1l2 = _combine([u01, u11, u21], _N1)
    x2l2 = _combine([u12, u22], _N2)
    return x0l2, x1l2, x2l2


# speedup vs baseline: 4.0505x; 1.0473x over previous
import functools

import jax
import jax.numpy as jnp
from jax import lax
from jax.experimental import pallas as pl
from jax.experimental.pallas import tpu as pltpu
from jax.experimental.pallas import tpu_sc as plsc

_N0, _N1, _N2, _D = 10000, 30000, 20000, 128
_NEG = 0.2
_NC = 2
_NS = 16
_NW = _NC * _NS
_CH1 = 256
_CH2 = 512
_EGRAN = _NW * max(_CH1, _CH2)


def _pad_len(n):
    return ((n + 1 + 255) // 256) * 256


def _pad_edges(e):
    return ((e + _EGRAN - 1) // _EGRAN) * _EGRAN


def _mm_body(x_ref, w_ref, ap_ref, m_ref, s_ref):
    m = jnp.dot(x_ref[...], w_ref[...], preferred_element_type=jnp.float32)
    m_ref[...] = m
    s_ref[...] = jnp.dot(m, ap_ref[...], preferred_element_type=jnp.float32)


def _dense(x, w, a):
    n = x.shape[0]
    blk = 1000
    ap = jnp.pad(jnp.concatenate([a[:_D], a[_D:]], axis=1), ((0, 0), (0, 6)))
    m, s = pl.pallas_call(
        _mm_body,
        grid=(n // blk,),
        in_specs=[pl.BlockSpec((blk, _D), lambda i: (i, 0)),
                  pl.BlockSpec((_D, _D), lambda i: (0, 0)),
                  pl.BlockSpec((_D, 8), lambda i: (0, 0))],
        out_specs=[pl.BlockSpec((blk, _D), lambda i: (i, 0)),
                   pl.BlockSpec((blk, 8), lambda i: (i, 0))],
        out_shape=[jax.ShapeDtypeStruct((n, _D), jnp.float32),
                   jax.ShapeDtypeStruct((n, 8), jnp.float32)],
    )(x, w, ap)
    return m, s[:, 0], s[:, 1]


@functools.lru_cache(maxsize=None)
def _make_scores(n_t_pad, n_s_pad, e_pad):
    iters = e_pad // (_NW * _CH1)
    mesh = plsc.VectorSubcoreMesh(core_axis_name="c", subcore_axis_name="s",
                                  num_cores=_NC)

    def body(st_hbm, ss_hbm, ii_hbm, jj_hbm, w_hbm, dt_hbm, ds_hbm,
             st_v, ss_v, dt_v, ds_v, iv, jv, wv, sem):
        del sem
        wid = lax.axis_index("s") * _NC + lax.axis_index("c")
        pltpu.sync_copy(st_hbm, st_v)
        pltpu.sync_copy(ss_hbm, ss_v)
        zero16 = jnp.zeros((16,), jnp.float32)

        def z_t(k, c):
            dt_v[pl.ds(k * 16, 16)] = zero16
            return c
        lax.fori_loop(0, n_t_pad // 16, z_t, 0)

        def z_s(k, c):
            ds_v[pl.ds(k * 16, 16)] = zero16
            return c
        lax.fori_loop(0, n_s_pad // 16, z_s, 0)

        def eloop(g, c):
            base = (g * _NW + wid) * _CH1
            pltpu.sync_copy(ii_hbm.at[pl.ds(base, _CH1)], iv)
            pltpu.sync_copy(jj_hbm.at[pl.ds(base, _CH1)], jv)

            def chunk(k, c2):
                i16 = iv[pl.ds(k * 16, 16)]
                j16 = jv[pl.ds(k * 16, 16)]
                e = plsc.load_gather(st_v, [i16]) + plsc.load_gather(ss_v, [j16])
                e = jnp.where(e >= 0, e, _NEG * e)
                w16 = jnp.exp(e)
                wv[pl.ds(k * 16, 16)] = w16
                plsc.addupdate_scatter(dt_v, [i16], w16)
                plsc.addupdate_scatter(ds_v, [j16], w16)
                return c2
            lax.fori_loop(0, _CH1 // 16, chunk, 0)
            pltpu.sync_copy(wv, w_hbm.at[pl.ds(base, _CH1)])
            return c
        lax.fori_loop(0, iters, eloop, 0)
        pltpu.sync_copy(dt_v, dt_hbm.at[wid])
        pltpu.sync_copy(ds_v, ds_hbm.at[wid])

    return pl.kernel(
        body, mesh=mesh,
        compiler_params=pltpu.CompilerParams(needs_layout_passes=False),
        out_type=[jax.ShapeDtypeStruct((e_pad,), jnp.float32),
                  jax.ShapeDtypeStruct((_NW, n_t_pad), jnp.float32),
                  jax.ShapeDtypeStruct((_NW, n_s_pad), jnp.float32)],
        scratch_types=[pltpu.VMEM((n_t_pad,), jnp.float32),
                       pltpu.VMEM((n_s_pad,), jnp.float32),
                       pltpu.VMEM((n_t_pad,), jnp.float32),
                       pltpu.VMEM((n_s_pad,), jnp.float32),
                       pltpu.VMEM((_CH1,), jnp.int32),
                       pltpu.VMEM((_CH1,), jnp.int32),
                       pltpu.VMEM((_CH1,), jnp.float32),
                       pltpu.SemaphoreType.DMA],
    )


@functools.lru_cache(maxsize=None)
def _make_aggregate(n_t_pad, n_s_pad, e_pad):
    del n_s_pad
    iters = e_pad // (_NW * _CH2)
    mesh = plsc.VectorSubcoreMesh(core_axis_name="c", subcore_axis_name="s",
                                  num_cores=_NC)

    def body(m0, m1, m2, m3, di_hbm, sj_hbm, w_hbm, zr_hbm,
             a0, a1, a2, a3, dv, sv, wv, rows, shacc, sem):
        sid = lax.axis_index("s")
        cid = lax.axis_index("c")
        wid = sid * _NC + cid
        msrc = (m0, m1, m2, m3)
        outs = (a0, a1, a2, a3)
        sl = n_t_pad // _NS
        for cc in range(4):
            @pl.when(sid == 0)
            def _():
                pltpu.sync_copy(zr_hbm, shacc)
            plsc.subcore_barrier()

            def eloop(g, c):
                base = (g * _NW + wid) * _CH2
                pltpu.sync_copy(di_hbm.at[pl.ds(base, _CH2)], dv)
                pltpu.sync_copy(sj_hbm.at[pl.ds(base, _CH2)], sv)
                pltpu.sync_copy(w_hbm.at[pl.ds(base, _CH2)], wv)
                pltpu.async_copy(msrc[cc].at[sv], rows, sem).wait()

                def scale(k, c2):
                    w16 = wv[pl.ds(k * 16, 16)]
                    for t in range(16):
                        wsc = w16[t]
                        e = k * 16 + t
                        rows[e, pl.ds(0, 16)] = rows[e, pl.ds(0, 16)] * wsc
                        rows[e, pl.ds(16, 16)] = rows[e, pl.ds(16, 16)] * wsc
                    return c2
                lax.fori_loop(0, _CH2 // 16, scale, 0)
                pltpu.sync_copy(rows, shacc.at[dv], add=True)
                return c
            lax.fori_loop(0, iters, eloop, 0)
            plsc.subcore_barrier()
            lo = sid * sl
            pltpu.sync_copy(shacc.at[pl.ds(lo, sl)],
                            outs[cc].at[cid, pl.ds(lo, sl)])
            plsc.subcore_barrier()

    return pl.kernel(
        body, mesh=mesh,
        compiler_params=pltpu.CompilerParams(needs_layout_passes=False,
                                             use_tc_tiling_on_sc=False),
        out_type=[jax.ShapeDtypeStruct((_NC, n_t_pad, 32), jnp.float32)] * 4,
        scratch_types=[pltpu.VMEM((_CH2,), jnp.int32),
                       pltpu.VMEM((_CH2,), jnp.int32),
                       pltpu.VMEM((_CH2,), jnp.float32),
                       pltpu.VMEM((_CH2, 32), jnp.float32),
                       pltpu.VMEM_SHARED((n_t_pad, 32), jnp.float32),
                       pltpu.SemaphoreType.DMA],
    )


def _edge_scores(st, ss, ii, jj, n_t, n_s):
    n_t_pad, n_s_pad = _pad_len(n_t), _pad_len(n_s)
    e_pad = _pad_edges(ii.shape[0])
    stp = jnp.pad(st, (0, n_t_pad - n_t))
    ssp = jnp.pad(ss, (0, n_s_pad - n_s))
    iip = jnp.pad(ii, (0, e_pad - ii.shape[0]), constant_values=n_t)
    jjp = jnp.pad(jj, (0, e_pad - jj.shape[0]), constant_values=n_s)
    w, dtp, dsp = _make_scores(n_t_pad, n_s_pad, e_pad)(stp, ssp, iip, jjp)
    return w, dtp, dsp, iip, jjp


def _edge_aggregate(msg, w, di, sj, n_t, n_s):
    n_t_pad, n_s_pad = _pad_len(n_t), _pad_len(n_s)
    mp = jnp.pad(msg, ((0, n_s_pad - n_s), (0, 0)))
    parts = [mp[:, k * 32:(k + 1) * 32] for k in range(4)]
    zr = jnp.zeros((n_t_pad, 32), jnp.float32)
    a0, a1, a2, a3 = _make_aggregate(n_t_pad, n_s_pad, di.shape[0])(
        parts[0], parts[1], parts[2], parts[3], di, sj, w, zr)
    acc_a = jnp.concatenate([a0[0], a1[0], a2[0], a3[0]], axis=1)[:n_t]
    acc_b = jnp.concatenate([a0[1], a1[1], a2[1], a3[1]], axis=1)[:n_t]
    return acc_a, acc_b


def _combine(terms, n):
    blk = 1000
    k = len(terms)

    def body(*refs):
        out = jnp.zeros((blk, _D), jnp.float32)
        for t in range(k):
            a = refs[3 * t][...] + refs[3 * t + 1][...]
            d = jnp.sum(refs[3 * t + 2][...], axis=1, keepdims=True)
            d = jnp.where(d == 0., 1., d)
            out = out + jnp.maximum(a / d, 0.)
        refs[-1][...] = out

    in_specs = []
    args = []
    for (acc_a, acc_b, dp) in terms:
        in_specs.append(pl.BlockSpec((blk, _D), lambda i: (i, 0)))
        in_specs.append(pl.BlockSpec((blk, _D), lambda i: (i, 0)))
        in_specs.append(pl.BlockSpec((blk, _NW), lambda i: (i, 0)))
        args.append(acc_a)
        args.append(acc_b)
        args.append(dp.T[:n])
    return pl.pallas_call(
        body,
        grid=(n // blk,),
        in_specs=in_specs,
        out_specs=pl.BlockSpec((blk, _D), lambda i: (i, 0)),
        out_shape=jax.ShapeDtypeStruct((n, _D), jnp.float32),
    )(*args)


def _hbs(x, idx, w, a, n):
    m, s_top, s_bot = _dense(x, w, a)
    ii, jj = idx[0], idx[1]
    wv, dtp, _, iip, jjp = _edge_scores(s_top, s_bot, ii, jj, n, n)
    acc_a, acc_b = _edge_aggregate(m, wv, iip, jjp, n, n)
    return acc_a, acc_b, dtp


def _hbns(x_s, x_t, idx, w_s, w_t, a, n_s, n_t):
    sm, sm_top, _ = _dense(x_s, w_s, a)
    tm, _, tm_bot = _dense(x_t, w_t, a)
    ii, jj = idx[0], idx[1]
    wv, dtp, dsp, iip, jjp = _edge_scores(tm_bot, sm_top, ii, jj, n_t, n_s)
    at_a, at_b = _edge_aggregate(sm, wv, iip, jjp, n_t, n_s)
    as_a, as_b = _edge_aggregate(tm, wv, jjp, iip, n_s, n_t)
    return (as_a, as_b, dsp), (at_a, at_b, dtp)


def kernel(x_0, x_1, x_2, adjacency_0, adjacency_1, coadjacency_2,
           incidence_1, incidence_2,
           w_hbs0_l1, a_hbs0_l1, ws_01_l1, wt_01_l1, a_01_l1,
           ws_12_l1, wt_12_l1, a_12_l1,
           w_hbs0_l2, a_hbs0_l2, ws_01_l2, wt_01_l2, a_01_l2,
           w_hbs1_l2, a_hbs1_l2, ws_12_l2, wt_12_l2, a_12_l2,
           w_hbs2_l2, a_hbs2_l2):
    t00 = _hbs(x_0, adjacency_0, w_hbs0_l1, a_hbs0_l1, _N0)
    t01, t10 = _hbns(x_1, x_0, incidence_1,
                     ws_01_l1, wt_01_l1, a_01_l1, _N1, _N0)
    t12, t21 = _hbns(x_2, x_1, incidence_2,
                     ws_12_l1, wt_12_l1, a_12_l1, _N2, _N1)
    x0l1 = _combine([t00, t10], _N0)
    x1l1 = _combine([t01, t21], _N1)
    x2l1 = _combine([t12], _N2)

    u00 = _hbs(x0l1, adjacency_0, w_hbs0_l2, a_hbs0_l2, _N0)
    u01, u10 = _hbns(x1l1, x0l1, incidence_1,
                     ws_01_l2, wt_01_l2, a_01_l2, _N1, _N0)
    u11 = _hbs(x1l1, adjacency_1, w_hbs1_l2, a_hbs1_l2, _N1)
    u12, u21 = _hbns(x2l1, x1l1, incidence_2,
                     ws_12_l2, wt_12_l2, a_12_l2, _N2, _N1)
    u22 = _hbs(x2l1, coadjacency_2, w_hbs2_l2, a_hbs2_l2, _N2)
    x0l2 = _combine([u00, u10], _N0)
    x1l2 = _combine([u01, u11, u21], _N1)
    x2l2 = _combine([u12, u22], _N2)
    return x0l2, x1l2, x2l2
